# baseline (speedup 1.0000x reference)
import dataclasses
import functools

import jax
import jax.numpy as jnp
from jax import lax
from jax.experimental import pallas as pl
from jax.experimental.pallas import tpu as pltpu
from jax.experimental.pallas import tpu_sc as plsc

B, L, D, MAX_LEN = 4, 2048, 1024, 2048
N = B * L
NUM_WORKERS = 32
ROWS_PER_WORKER = N // NUM_WORKERS
CHUNK = 8
NCHUNK = ROWS_PER_WORKER // CHUNK
RING = 6
MAIN = (NCHUNK // RING) * RING
TAIL = NCHUNK - MAIN
LANES = 16
UNROLL = 8
CT = D // 128
NSEG = CHUNK * CT

_CP = pltpu.CompilerParams()
if "needs_layout_passes" in pltpu.CompilerParams.__dataclass_fields__:
    _CP = dataclasses.replace(_CP, needs_layout_passes=False)


@functools.partial(
    pl.kernel,
    out_type=jax.ShapeDtypeStruct((N * D,), jnp.float32),
    mesh=plsc.VectorSubcoreMesh(core_axis_name="c", subcore_axis_name="s"),
    compiler_params=_CP,
    scratch_types=(
        [pltpu.VMEM((ROWS_PER_WORKER,), jnp.int32),
         pltpu.VMEM((ROWS_PER_WORKER,), jnp.float32)]
        + [pltpu.VMEM((CHUNK * D,), jnp.float32)] * RING
        + [pltpu.VMEM((NSEG, 128), jnp.float32)] * RING
        + [pltpu.VMEM((NSEG,), jnp.int32)] * RING
        + [pltpu.SemaphoreType.DMA] * (3 * RING + 1)
    ),
)
def _sc_masked_pe(x_hbm, mask_hbm, pos_hbm, pe_hbm, out_hbm,
                  pos_v, msk_v, *scratch):
    xb_ = scratch[0:RING]
    pb_ = scratch[RING:2 * RING]
    ib_ = scratch[2 * RING:3 * RING]
    sg_ = scratch[3 * RING:4 * RING]
    sx_ = scratch[4 * RING:5 * RING]
    ss_ = scratch[5 * RING:6 * RING]
    st_sem = scratch[6 * RING]
    bufs = tuple(zip(xb_, pb_, ib_, sg_, sx_, ss_))

    wid = lax.axis_index("s") * 2 + lax.axis_index("c")
    base = wid * ROWS_PER_WORKER

    cp_pos = pltpu.async_copy(
        pos_hbm.at[pl.ds(base, ROWS_PER_WORKER)], pos_v, st_sem)
    cp_msk = pltpu.async_copy(
        mask_hbm.at[pl.ds(base, ROWS_PER_WORKER)], msk_v, st_sem)
    cp_pos.wait()
    cp_msk.wait()

    it = lax.iota(jnp.int32, LANES)
    dst0 = ((it >> 3) << 3) + (it & 7)

    def issue(c, buf):
        x_b, pe_b, ix_b, sg, sx, _ = buf
        off = pl.multiple_of(c * CHUNK, CHUNK)
        p = plsc.load_gather(pos_v, [off + (it & 7)])
        seg0 = ((p >> 3) << 6) + (p & 7) + ((it >> 3) << 3)
        for h in range(CT // 2):
            plsc.store_scatter(ix_b, [dst0 + (h << 4)], seg0 + (h << 4))
        pltpu.async_copy(pe_hbm.at[ix_b], pe_b, sg)
        pltpu.async_copy(
            x_hbm.at[pl.ds(pl.multiple_of((base + off) * D, CHUNK * D),
                           CHUNK * D)], x_b, sx)

    def consume(c, b):
        x_b, pe_b, ix_b, sg, sx, ss = bufs[b]
        off = pl.multiple_of(c * CHUNK, CHUNK)
        hoff = pl.multiple_of((base + off) * D, CHUNK * D)
        pltpu.make_async_copy(pe_hbm.at[pl.ds(0, NSEG)], pe_b, sg).wait()
        pltpu.make_async_copy(x_hbm.at[pl.ds(0, CHUNK * D)], x_b, sx).wait()

        @plsc.parallel_loop(0, CHUNK)
        def _row(r):
            m = plsc.load_gather(
                msk_v, [jnp.full((LANES,), off + r, jnp.int32)])
            rbase = (r & 7) << 7
            prow0 = r & 7

            @plsc.parallel_loop(0, CT)
            def _ctile(ct):
                xtb = rbase + (ct << 10)
                prow = prow0 + (ct << 3)

                @plsc.parallel_loop(0, 128, LANES, unroll=UNROLL)
                def _col(j):
                    xs = pl.ds(xtb + j, LANES)
                    x_b[xs] = x_b[xs] + m * pe_b[prow, pl.ds(j, LANES)]

        pltpu.async_copy(x_b, out_hbm.at[pl.ds(hoff, CHUNK * D)], ss)

        @pl.when(c + RING < NCHUNK)
        def _refill():
            pltpu.make_async_copy(
                x_b, out_hbm.at[pl.ds(0, CHUNK * D)], ss).wait()
            issue(c + RING, bufs[b])

    for b in range(RING):
        issue(b, bufs[b])

    @pl.loop(0, MAIN, step=RING)
    def _ring(i):
        for b in range(RING):
            consume(i + b, b)

    for b in range(TAIL):
        consume(MAIN + b, b)

    for b in range(RING):
        pltpu.make_async_copy(
            bufs[b][0], out_hbm.at[pl.ds(0, CHUNK * D)], bufs[b][5]).wait()


@jax.jit
def kernel(x, source_mask, positions, positional_encoding):
    xb = jnp.transpose(x.reshape(N // 8, 8, CT, 128), (0, 2, 1, 3)).reshape(N * D)
    peb = jnp.transpose(
        positional_encoding.reshape(MAX_LEN // 8, 8, CT, 128),
        (0, 2, 1, 3)).reshape(MAX_LEN * CT, 128)
    mask = source_mask.reshape(N).astype(jnp.float32)
    pos = positions.reshape(N).astype(jnp.int32)
    out = _sc_masked_pe(xb, mask, pos, peb)
    return jnp.transpose(
        out.reshape(N // 8, CT, 8, 128), (0, 2, 1, 3)).reshape(B, L, D)


# speedup vs baseline: 1.0051x; 1.0051x over previous
import dataclasses
import functools

import jax
import jax.numpy as jnp
from jax import lax
from jax.experimental import pallas as pl
from jax.experimental.pallas import tpu as pltpu
from jax.experimental.pallas import tpu_sc as plsc

B, L, D, MAX_LEN = 4, 2048, 1024, 2048
N = B * L
NUM_WORKERS = 32
ROWS_PER_WORKER = N // NUM_WORKERS
CHUNK = 8
NCHUNK = ROWS_PER_WORKER // CHUNK
RING = 4
LANES = 16
UNROLL = 8
CT = D // 128
NSEG = CHUNK * CT

_CP = pltpu.CompilerParams()
if "needs_layout_passes" in pltpu.CompilerParams.__dataclass_fields__:
    _CP = dataclasses.replace(_CP, needs_layout_passes=False)


@functools.partial(
    pl.kernel,
    out_type=jax.ShapeDtypeStruct((N * D,), jnp.float32),
    mesh=plsc.VectorSubcoreMesh(core_axis_name="c", subcore_axis_name="s"),
    compiler_params=_CP,
    scratch_types=(
        [pltpu.VMEM((ROWS_PER_WORKER,), jnp.int32),
         pltpu.VMEM((ROWS_PER_WORKER,), jnp.float32)]
        + [pltpu.VMEM((CHUNK * D,), jnp.float32)] * RING
        + [pltpu.VMEM((NSEG, 128), jnp.float32)] * RING
        + [pltpu.VMEM((NSEG,), jnp.int32)] * RING
        + [pltpu.SemaphoreType.DMA] * (3 * RING + 1)
    ),
)
def _sc_masked_pe(x_hbm, mask_hbm, pos_hbm, pe_hbm, out_hbm,
                  pos_v, msk_v, *scratch):
    xb_ = scratch[0:RING]
    pb_ = scratch[RING:2 * RING]
    ib_ = scratch[2 * RING:3 * RING]
    sg_ = scratch[3 * RING:4 * RING]
    sx_ = scratch[4 * RING:5 * RING]
    ss_ = scratch[5 * RING:6 * RING]
    st_sem = scratch[6 * RING]
    bufs = tuple(zip(xb_, pb_, ib_, sg_, sx_, ss_))

    wid = lax.axis_index("s") * 2 + lax.axis_index("c")
    base = wid * ROWS_PER_WORKER

    cp_pos = pltpu.async_copy(
        pos_hbm.at[pl.ds(base, ROWS_PER_WORKER)], pos_v, st_sem)
    cp_msk = pltpu.async_copy(
        mask_hbm.at[pl.ds(base, ROWS_PER_WORKER)], msk_v, st_sem)
    cp_pos.wait()
    cp_msk.wait()

    it = lax.iota(jnp.int32, LANES)
    dst0 = ((it >> 3) << 3) + (it & 7)

    def issue(c, buf):
        x_b, pe_b, ix_b, sg, sx, _ = buf
        off = pl.multiple_of(c * CHUNK, CHUNK)
        p = plsc.load_gather(pos_v, [off + (it & 7)])
        seg0 = ((p >> 3) << 6) + (p & 7) + ((it >> 3) << 3)
        for h in range(CT // 2):
            plsc.store_scatter(ix_b, [dst0 + (h << 4)], seg0 + (h << 4))
        pltpu.async_copy(pe_hbm.at[ix_b], pe_b, sg)
        pltpu.async_copy(
            x_hbm.at[pl.ds(pl.multiple_of((base + off) * D, CHUNK * D),
                           CHUNK * D)], x_b, sx)

    for b in range(RING):
        issue(b, bufs[b])

    @pl.loop(0, NCHUNK, step=RING)
    def _ring(i):
        for b in range(RING):
            c = i + b
            x_b, pe_b, ix_b, sg, sx, ss = bufs[b]
            off = pl.multiple_of(c * CHUNK, CHUNK)
            hoff = pl.multiple_of((base + off) * D, CHUNK * D)
            pltpu.make_async_copy(pe_hbm.at[pl.ds(0, NSEG)], pe_b, sg).wait()
            pltpu.make_async_copy(x_hbm.at[pl.ds(0, CHUNK * D)], x_b, sx).wait()

            @plsc.parallel_loop(0, CHUNK)
            def _row(r):
                m = plsc.load_gather(
                    msk_v, [jnp.full((LANES,), off + r, jnp.int32)])
                rbase = (r & 7) << 7
                prow0 = r & 7

                @plsc.parallel_loop(0, CT)
                def _ctile(ct):
                    xtb = rbase + (ct << 10)
                    prow = prow0 + (ct << 3)

                    @plsc.parallel_loop(0, 128, LANES, unroll=UNROLL)
                    def _col(j):
                        xs = pl.ds(xtb + j, LANES)
                        x_b[xs] = x_b[xs] + m * pe_b[prow, pl.ds(j, LANES)]

            pltpu.async_copy(x_b, out_hbm.at[pl.ds(hoff, CHUNK * D)], ss)

            @pl.when(c + RING < NCHUNK)
            def _refill():
                pltpu.make_async_copy(
                    x_b, out_hbm.at[pl.ds(0, CHUNK * D)], ss).wait()
                issue(c + RING, bufs[b])

    for b in range(RING):
        pltpu.make_async_copy(
            bufs[b][0], out_hbm.at[pl.ds(0, CHUNK * D)], bufs[b][5]).wait()


@jax.jit
def kernel(x, source_mask, positions, positional_encoding):
    xb = jnp.transpose(x.reshape(N // 8, 8, CT, 128), (0, 2, 1, 3)).reshape(N * D)
    peb = jnp.transpose(
        positional_encoding.reshape(MAX_LEN // 8, 8, CT, 128),
        (0, 2, 1, 3)).reshape(MAX_LEN * CT, 128)
    mask = source_mask.reshape(N).astype(jnp.float32)
    pos = positions.reshape(N).astype(jnp.int32)
    out = _sc_masked_pe(xb, mask, pos, peb)
    return jnp.transpose(
        out.reshape(N // 8, CT, 8, 128), (0, 2, 1, 3)).reshape(B, L, D)
